# baseline (speedup 1.0000x reference)
import dataclasses
import functools

import jax
import jax.numpy as jnp
from jax import lax
from jax.experimental import pallas as pl
from jax.experimental.pallas import tpu as pltpu
from jax.experimental.pallas import tpu_sc as plsc


_LANES = 16


def _sc_trilinear(x, flat_lut, dim):
    B, C, H, W = x.shape
    ncores, nsub = 2, 16
    nw = ncores * nsub
    workers_per_batch = nw // B
    rows_per_worker = H // workers_per_batch
    tiles_h = rows_per_worker // 8
    tiles_w = W // 128
    ntiles = tiles_h * tiles_w
    lut_pad = flat_lut.shape[0]
    dim2 = dim * dim
    dim3 = dim2 * dim
    scale = float(dim - 1)

    mesh = plsc.VectorSubcoreMesh(core_axis_name="c", subcore_axis_name="s")

    cp = pltpu.CompilerParams(use_tc_tiling_on_sc=True)
    if "needs_layout_passes" in pltpu.CompilerParams.__dataclass_fields__:
        cp = dataclasses.replace(cp, needs_layout_passes=False)

    tile_t = pltpu.VMEM((8, 128), jnp.float32)

    @functools.partial(
        pl.kernel,
        compiler_params=cp,
        out_type=jax.ShapeDtypeStruct((B, C, H, W), jnp.float32),
        mesh=mesh,
        scratch_types=[
            pltpu.VMEM((lut_pad,), jnp.float32),
            tile_t, tile_t, tile_t, tile_t, tile_t, tile_t,
            tile_t, tile_t, tile_t, tile_t, tile_t, tile_t,
            pltpu.SemaphoreType.DMA,
            pltpu.SemaphoreType.DMA,
            pltpu.SemaphoreType.DMA,
            pltpu.SemaphoreType.DMA,
            pltpu.SemaphoreType.DMA,
        ],
    )
    def sc_kernel(
        x_hbm, lut_hbm, o_hbm,
        lut_v,
        i00, i01, i02, i10, i11, i12,
        o00, o01, o02, o10, o11, o12,
        lsem, isem0, isem1, osem0, osem1,
    ):
        ins = ((i00, i01, i02), (i10, i11, i12))
        outs = ((o00, o01, o02), (o10, o11, o12))

        wid = lax.axis_index("s") * ncores + lax.axis_index("c")
        batch = wid // workers_per_batch
        row0 = (wid % workers_per_batch) * rows_per_worker

        pltpu.async_copy(lut_hbm, lut_v, lsem)

        def tile_slices(t):
            h0 = row0 + (t // tiles_w) * 8
            w0 = (t % tiles_w) * 128
            return h0, w0

        def copy_in(bufs, t, sem):
            h0, w0 = tile_slices(t)
            for c in range(3):
                pltpu.async_copy(
                    x_hbm.at[batch, c, pl.ds(h0, 8), pl.ds(w0, 128)],
                    bufs[c],
                    sem,
                )

        def wait_in(bufs, sem):
            for c in range(3):
                pltpu.make_async_copy(
                    x_hbm.at[batch, c, pl.ds(row0, 8), pl.ds(0, 128)],
                    bufs[c],
                    sem,
                ).wait()

        def copy_out(bufs, t, sem):
            h0, w0 = tile_slices(t)
            for c in range(3):
                pltpu.async_copy(
                    bufs[c],
                    o_hbm.at[batch, c, pl.ds(h0, 8), pl.ds(w0, 128)],
                    sem,
                )

        def wait_out(bufs, sem):
            for c in range(3):
                pltpu.make_async_copy(
                    bufs[c],
                    o_hbm.at[batch, c, pl.ds(row0, 8), pl.ds(0, 128)],
                    sem,
                ).wait()

        def compute(ibufs, obufs):
            @plsc.parallel_loop(0, 8, step=1, unroll=2)
            def _row(rw):
                for cg in range(128 // _LANES):
                    sl = pl.ds(cg * _LANES, _LANES)
                    r = ibufs[0][rw, sl]
                    g = ibufs[1][rw, sl]
                    b = ibufs[2][rw, sl]

                    def bin_of(v):
                        vs = v * scale
                        vc = jnp.minimum(jnp.maximum(vs, 0.0), scale - 0.5)
                        vi = vc.astype(jnp.int32)
                        vd = vs - vi.astype(jnp.float32)
                        return vi, vd

                    r_id, r_d = bin_of(r)
                    g_id, g_d = bin_of(g)
                    b_id, b_d = bin_of(b)
                    base = b_id * dim2 + g_id * dim + r_id

                    for c in range(3):
                        cbase = base + c * dim3

                        def corner(db, dg, dr):
                            idx = cbase + (db * dim2 + dg * dim + dr)
                            return plsc.load_gather(lut_v, [idx])

                        m00 = corner(0, 0, 0)
                        m00 = m00 + (corner(0, 0, 1) - m00) * r_d
                        m01 = corner(0, 1, 0)
                        m01 = m01 + (corner(0, 1, 1) - m01) * r_d
                        m10 = corner(1, 0, 0)
                        m10 = m10 + (corner(1, 0, 1) - m10) * r_d
                        m11 = corner(1, 1, 0)
                        m11 = m11 + (corner(1, 1, 1) - m11) * r_d
                        n0 = m00 + (m01 - m00) * g_d
                        n1 = m10 + (m11 - m10) * g_d
                        obufs[c][rw, sl] = n0 + (n1 - n0) * b_d

        copy_in(ins[0], 0, isem0)
        copy_in(ins[1], 1, isem1)
        pltpu.make_async_copy(lut_hbm, lut_v, lsem).wait()

        @pl.loop(0, ntiles, step=2)
        def _window(k):
            wait_in(ins[0], isem0)

            @pl.when(k >= 2)
            def _():
                wait_out(outs[0], osem0)

            compute(ins[0], outs[0])
            copy_out(outs[0], k, osem0)

            @pl.when(k + 2 < ntiles)
            def _():
                copy_in(ins[0], k + 2, isem0)

            wait_in(ins[1], isem1)

            @pl.when(k >= 2)
            def _():
                wait_out(outs[1], osem1)

            compute(ins[1], outs[1])
            copy_out(outs[1], k + 1, osem1)

            @pl.when(k + 3 < ntiles)
            def _():
                copy_in(ins[1], k + 3, isem1)

        wait_out(outs[0], osem0)
        wait_out(outs[1], osem1)

    return sc_kernel(x, flat_lut)


def kernel(x, LUT):
    dim = LUT.shape[1]
    flat_lut = LUT.reshape(-1)
    lut_pad = ((flat_lut.shape[0] + 15) // 16) * 16
    flat_lut = jnp.pad(flat_lut, (0, lut_pad - flat_lut.shape[0]))
    return _sc_trilinear(x, flat_lut, dim)
